# baseline (speedup 1.0000x reference)
import functools
import math

import jax
import jax.numpy as jnp
from jax import lax
from jax.experimental import pallas as pl
from jax.experimental.pallas import tpu as pltpu
from jax.experimental.pallas import tpu_sc as plsc

_NC = 2
_NS = 16
_NW = _NC * _NS
_LANES = 16


def _sin_pe(seq_len, d_model):
    pos = jnp.arange(seq_len, dtype=jnp.float32)[:, None]
    i = jnp.arange(d_model, dtype=jnp.float32)[None, :]
    angle_rates = jnp.power(10000.0, -(2.0 * jnp.floor(i / 2.0)) / d_model)
    angles = pos * angle_rates
    even = (jnp.arange(d_model) % 2) == 0
    return jnp.where(even[None, :], jnp.sin(angles), jnp.cos(angles))


@functools.partial(jax.jit, static_argnums=(3, 4))
def _embed_lookup(idx, pe, table, S, D):
    B_flat = idx.shape[0]
    b_per_w = B_flat // _NW
    n_chunks = b_per_w // S
    scale = float(math.sqrt(D))
    s0 = (S // 2) & ~7
    s1 = S - s0

    mesh = plsc.VectorSubcoreMesh(core_axis_name="c", subcore_axis_name="s")

    @functools.partial(
        pl.kernel,
        mesh=mesh,
        compiler_params=pltpu.CompilerParams(use_tc_tiling_on_sc=False),
        out_type=jax.ShapeDtypeStruct((B_flat, D), jnp.float32),
        scratch_types=[
            pltpu.VMEM((S,), jnp.int32),
            pltpu.VMEM((S, D), jnp.float32),
            pltpu.VMEM((S, D), jnp.float32),
            pltpu.SemaphoreType.DMA,
        ],
    )
    def k(idx_hbm, pe_hbm, table_hbm, out_hbm, idx_v, rows_v, pe_v, sem):
        wid = lax.axis_index("s") * _NC + lax.axis_index("c")
        base = wid * b_per_w
        pltpu.sync_copy(pe_hbm, pe_v)

        def chunk(ci, carry):
            off = base + ci * S
            pltpu.sync_copy(idx_hbm.at[pl.ds(off, S)], idx_v)
            cp0 = pltpu.async_copy(
                table_hbm.at[idx_v.at[pl.ds(0, s0)]], rows_v.at[pl.ds(0, s0)], sem)
            cp1 = pltpu.async_copy(
                table_hbm.at[idx_v.at[pl.ds(s0, s1)]], rows_v.at[pl.ds(s0, s1)], sem)
            cp0.wait()
            cp1.wait()

            def row(r, c):
                for j in range(D // _LANES):
                    sl = pl.ds(j * _LANES, _LANES)
                    rows_v[r, sl] = rows_v[r, sl] * scale + pe_v[r, sl]
                return c

            lax.fori_loop(0, S, row, 0)
            pltpu.sync_copy(rows_v, out_hbm.at[pl.ds(off, S)])
            return carry

        lax.fori_loop(0, n_chunks, chunk, 0)

    return k(idx, pe, table)


def kernel(input, table):
    Bb, S = input.shape
    V, D = table.shape
    idx = input.reshape(-1)
    if idx.dtype != jnp.int32:
        idx = idx.astype(jnp.int32)
    pe = _sin_pe(S, D).astype(table.dtype)
    out = _embed_lookup(idx, pe, table, S, D)
    return out.reshape(Bb, S, D)


# speedup vs baseline: 1.2234x; 1.2234x over previous
import functools
import math

import jax
import jax.numpy as jnp
from jax import lax
from jax.experimental import pallas as pl
from jax.experimental.pallas import tpu as pltpu
from jax.experimental.pallas import tpu_sc as plsc

_NC = 2
_NS = 16
_NW = _NC * _NS
_LANES = 16


def _sin_pe(seq_len, d_model):
    pos = jnp.arange(seq_len, dtype=jnp.float32)[:, None]
    i = jnp.arange(d_model, dtype=jnp.float32)[None, :]
    angle_rates = jnp.power(10000.0, -(2.0 * jnp.floor(i / 2.0)) / d_model)
    angles = pos * angle_rates
    even = (jnp.arange(d_model) % 2) == 0
    return jnp.where(even[None, :], jnp.sin(angles), jnp.cos(angles))


@functools.partial(jax.jit, static_argnums=(3, 4))
def _embed_lookup(idx, pe, table, S, D):
    B_flat = idx.shape[0]
    b_per_w = B_flat // _NW
    n_chunks = b_per_w // S
    scale = float(math.sqrt(D))
    s0 = (S // 2) & ~7
    s1 = S - s0

    mesh = plsc.VectorSubcoreMesh(core_axis_name="c", subcore_axis_name="s")
    NBUF = 4
    AHEAD = 2

    @functools.partial(
        pl.kernel,
        mesh=mesh,
        compiler_params=pltpu.CompilerParams(use_tc_tiling_on_sc=False),
        out_type=jax.ShapeDtypeStruct((B_flat, D), jnp.float32),
        scratch_types=[
            pltpu.VMEM((b_per_w,), jnp.int32),
            [pltpu.VMEM((S, D), jnp.float32) for _ in range(NBUF)],
            pltpu.VMEM((S, D), jnp.float32),
            pltpu.SemaphoreType.DMA,
            pltpu.SemaphoreType.DMA,
        ],
    )
    def k(idx_hbm, pe_hbm, table_hbm, out_hbm, idx_v, rows, pe_v, gsem, osem):
        wid = lax.axis_index("s") * _NC + lax.axis_index("c")
        base = wid * b_per_w
        pltpu.sync_copy(pe_hbm, pe_v)
        pltpu.sync_copy(idx_hbm.at[pl.ds(base, b_per_w)], idx_v)

        def gather_start(ci, buf):
            o = ci * S
            pltpu.async_copy(
                table_hbm.at[idx_v.at[pl.ds(o, s0)]], buf.at[pl.ds(0, s0)], gsem)
            pltpu.async_copy(
                table_hbm.at[idx_v.at[pl.ds(o + s0, s1)]], buf.at[pl.ds(s0, s1)], gsem)

        def gather_wait(buf):
            pltpu.make_async_copy(
                table_hbm.at[idx_v.at[pl.ds(0, s0)]], buf.at[pl.ds(0, s0)], gsem).wait()
            pltpu.make_async_copy(
                table_hbm.at[idx_v.at[pl.ds(s0, s1)]], buf.at[pl.ds(s0, s1)], gsem).wait()

        def out_wait(buf):
            pltpu.make_async_copy(buf, out_hbm.at[pl.ds(0, S)], osem).wait()

        for ci in range(AHEAD):
            gather_start(ci, rows[ci % NBUF])

        def outer(gi, carry):
            for b in range(NBUF):
                ci = gi * NBUF + b
                buf = rows[b]
                nxt = rows[(b + AHEAD) % NBUF]

                @pl.when(ci + AHEAD < n_chunks)
                def _issue():
                    @pl.when(ci + AHEAD >= NBUF)
                    def _wait_out():
                        out_wait(nxt)
                    gather_start(ci + AHEAD, nxt)

                gather_wait(buf)

                def row(r, c):
                    for j in range(D // _LANES):
                        sl = pl.ds(j * _LANES, _LANES)
                        buf[r, sl] = buf[r, sl] * scale + pe_v[r, sl]
                    return c

                lax.fori_loop(0, S, row, 0)
                pltpu.async_copy(buf, out_hbm.at[pl.ds(base + ci * S, S)], osem)
            return carry

        lax.fori_loop(0, n_chunks // NBUF, outer, 0)
        for b in range(NBUF):
            out_wait(rows[b])

    return k(idx, pe, table)


def kernel(input, table):
    Bb, S = input.shape
    V, D = table.shape
    idx = input.reshape(-1)
    if idx.dtype != jnp.int32:
        idx = idx.astype(jnp.int32)
    pe = _sin_pe(S, D).astype(table.dtype)
    out = _embed_lookup(idx, pe, table, S, D)
    return out.reshape(Bb, S, D)
